# baseline (speedup 1.0000x reference)
import jax
import jax.numpy as jnp
from jax.experimental import pallas as pl
from jax.experimental.pallas import tpu as pltpu

_N = 4096
_D = 128
_BM = 1024
_BK = 1024
_NI = _N // _BM
_NK = _N // _BK


def _proj_body(x_ref, wi_ref, ws_ref, hi_ref, hs_ref):
    hi_ref[...] = jnp.dot(x_ref[...], wi_ref[...],
                          preferred_element_type=jnp.float32)
    hs_ref[...] = jnp.dot(x_ref[...], ws_ref[...],
                          preferred_element_type=jnp.float32)


def _agg_body(hi_ref, hs_ref, l_ref, u_ref, out_ref, acc_ref):
    k = pl.program_id(1)

    @pl.when(k == 0)
    def _():
        acc_ref[...] = jnp.zeros_like(acc_ref)

    hi = hi_ref[pl.ds(k * _BK, _BK), :]
    hs = hs_ref[pl.ds(k * _BK, _BK), :]
    acc_ref[...] += (
        jnp.dot(l_ref[...], hi, preferred_element_type=jnp.float32)
        + jnp.dot(u_ref[...], hs, preferred_element_type=jnp.float32))

    @pl.when(k == _NK - 1)
    def _():
        out_ref[...] = jnp.maximum(acc_ref[...], 0.0)


def kernel(x, lower_neighborhood, upper_neighborhood, W_irr, W_sol):
    h_irr, h_sol = pl.pallas_call(
        _proj_body,
        out_shape=(jax.ShapeDtypeStruct((_N, _D), jnp.float32),
                   jax.ShapeDtypeStruct((_N, _D), jnp.float32)),
    )(x, W_irr, W_sol)

    return pl.pallas_call(
        _agg_body,
        grid=(_NI, _NK),
        in_specs=[
            pl.BlockSpec((_N, _D), lambda i, k: (0, 0)),
            pl.BlockSpec((_N, _D), lambda i, k: (0, 0)),
            pl.BlockSpec((_BM, _BK), lambda i, k: (i, k)),
            pl.BlockSpec((_BM, _BK), lambda i, k: (i, k)),
        ],
        out_specs=pl.BlockSpec((_BM, _D), lambda i, k: (i, 0)),
        out_shape=jax.ShapeDtypeStruct((_N, _D), jnp.float32),
        scratch_shapes=[
            pltpu.VMEM((_BM, _D), jnp.float32),
        ],
        compiler_params=pltpu.CompilerParams(
            dimension_semantics=("parallel", "arbitrary")),
    )(h_irr, h_sol, lower_neighborhood, upper_neighborhood)


# speedup vs baseline: 1.0915x; 1.0915x over previous
import jax
import jax.numpy as jnp
from jax.experimental import pallas as pl
from jax.experimental.pallas import tpu as pltpu

_N = 4096
_D = 128
_BM = 1024
_BK = 1024
_NI = _N // _BM
_NK = _N // _BK


def _body(x_ref, wi_ref, ws_ref, l_ref, u_ref, out_ref, acc_ref, hi_ref, hs_ref):
    i = pl.program_id(0)
    k = pl.program_id(1)

    @pl.when(i == 0)
    def _():
        xb = x_ref[pl.ds(k * _BK, _BK), :]
        hi_ref[pl.ds(k * _BK, _BK), :] = jnp.dot(
            xb, wi_ref[...], preferred_element_type=jnp.float32)
        hs_ref[pl.ds(k * _BK, _BK), :] = jnp.dot(
            xb, ws_ref[...], preferred_element_type=jnp.float32)

    @pl.when(k == 0)
    def _():
        acc_ref[...] = jnp.zeros_like(acc_ref)

    hi = hi_ref[pl.ds(k * _BK, _BK), :]
    hs = hs_ref[pl.ds(k * _BK, _BK), :]
    acc_ref[...] += (
        jnp.dot(l_ref[...], hi, preferred_element_type=jnp.float32)
        + jnp.dot(u_ref[...], hs, preferred_element_type=jnp.float32))

    @pl.when(k == _NK - 1)
    def _():
        out_ref[...] = jnp.maximum(acc_ref[...], 0.0)


def kernel(x, lower_neighborhood, upper_neighborhood, W_irr, W_sol):
    return pl.pallas_call(
        _body,
        grid=(_NI, _NK),
        in_specs=[
            pl.BlockSpec((_N, _D), lambda i, k: (0, 0)),
            pl.BlockSpec((_D, _D), lambda i, k: (0, 0)),
            pl.BlockSpec((_D, _D), lambda i, k: (0, 0)),
            pl.BlockSpec((_BM, _BK), lambda i, k: (i, k)),
            pl.BlockSpec((_BM, _BK), lambda i, k: (i, k)),
        ],
        out_specs=pl.BlockSpec((_BM, _D), lambda i, k: (i, 0)),
        out_shape=jax.ShapeDtypeStruct((_N, _D), jnp.float32),
        scratch_shapes=[
            pltpu.VMEM((_BM, _D), jnp.float32),
            pltpu.VMEM((_N, _D), jnp.float32),
            pltpu.VMEM((_N, _D), jnp.float32),
        ],
        compiler_params=pltpu.CompilerParams(
            dimension_semantics=("arbitrary", "arbitrary")),
    )(x, W_irr, W_sol, lower_neighborhood, upper_neighborhood)
